# Initial kernel scaffold; baseline (speedup 1.0000x reference)
import jax
import jax.numpy as jnp
from jax.experimental import pallas as pl


def kernel(x, edge_index, batch, target, embs, W1, b1, W2, b2, W3, b3, Wg1, bg1, Wg2, bg2, emb_xt, k_xt, bc_xt, W_xt, b_xt, Wp, a_p, Wfg, bfg, Wf1, bf1, Wf2, bf2, Wo, bo):
    raise NotImplementedError("write your pallas kernel here")


# speedup vs baseline: 1.0904x; 1.0904x over previous
import functools
import jax
import jax.numpy as jnp
from jax.experimental import pallas as pl
from jax.experimental.pallas import tpu as pltpu

N = 10000
E = 320000
B = 128
BLK = 512
E2 = E + N
NBLK = (E2 + BLK - 1) // BLK
E2P = NBLK * BLK
NPAD = ((N - 1 + BLK + 511) // 512) * 512


def _scatter_kern(dmins_ref, vals_ref, coeff_ref, ds_ref, out_ref):
    j = pl.program_id(0)

    @pl.when(j == 0)
    def _():
        out_ref[:, :] = jnp.zeros_like(out_ref)

    dmin = (dmins_ref[j] // 8) * 8
    ds = ds_ref[0, :, :]
    R = BLK + 8
    rows = jax.lax.broadcasted_iota(jnp.int32, (R, BLK), 0) + dmin
    ohT = (rows == ds).astype(jnp.float32)
    msg = vals_ref[:, :] * coeff_ref[:, :]
    part = jnp.dot(ohT, msg, preferred_element_type=jnp.float32)
    out_ref[pl.ds(dmin, R), :] += part


def _scatter(dmins, vals, coeff, ds3, F):
    grid_spec = pltpu.PrefetchScalarGridSpec(
        num_scalar_prefetch=1,
        grid=(NBLK,),
        in_specs=[
            pl.BlockSpec((BLK, F), lambda j, d: (j, 0)),
            pl.BlockSpec((BLK, 1), lambda j, d: (j, 0)),
            pl.BlockSpec((1, 1, BLK), lambda j, d: (j, 0, 0)),
        ],
        out_specs=pl.BlockSpec((NPAD, F), lambda j, d: (0, 0)),
    )
    return pl.pallas_call(
        _scatter_kern,
        grid_spec=grid_spec,
        out_shape=jax.ShapeDtypeStruct((NPAD, F), jnp.float32),
    )(dmins, vals, coeff, ds3)


def _mm_kern(x_ref, w_ref, o_ref):
    o_ref[:, :] = jnp.dot(x_ref[:, :], w_ref[:, :],
                          preferred_element_type=jnp.float32)


def _mm(x, w):
    return pl.pallas_call(
        _mm_kern,
        out_shape=jax.ShapeDtypeStruct((x.shape[0], w.shape[1]), jnp.float32),
    )(x, w)


def _mm_bias_relu_kern(x_ref, b_ref, w_ref, o_ref):
    act = jnp.maximum(x_ref[:, :] + b_ref[:, :], 0.0)
    o_ref[:, :] = jnp.dot(act, w_ref[:, :],
                          preferred_element_type=jnp.float32)


def _mm_bias_relu(x, b_row, w):
    return pl.pallas_call(
        _mm_bias_relu_kern,
        out_shape=jax.ShapeDtypeStruct((x.shape[0], w.shape[1]), jnp.float32),
    )(x, b_row, w)


def _conv_kern(et_ref, k_ref, bc_ref, o_ref):
    acc = jnp.zeros((32, 121), jnp.float32)
    for t in range(8):
        kt = k_ref[:, :, t]
        sl = et_ref[0, :, t:t + 121]
        acc = acc + jnp.dot(kt, sl, preferred_element_type=jnp.float32)
    o_ref[0, :, :] = acc + bc_ref[:, :]


def _conv(et, k_xt, bc_col):
    return pl.pallas_call(
        _conv_kern,
        grid=(B,),
        in_specs=[
            pl.BlockSpec((1, 1000, 128), lambda b: (b, 0, 0)),
            pl.BlockSpec((32, 1000, 8), lambda b: (0, 0, 0)),
            pl.BlockSpec((32, 1), lambda b: (0, 0)),
        ],
        out_specs=pl.BlockSpec((1, 32, 121), lambda b: (b, 0, 0)),
        out_shape=jax.ShapeDtypeStruct((B, 32, 121), jnp.float32),
    )(et, k_xt, bc_col)


def _head_kern(agg3_ref, batch_ref, b3_ref, wg1_ref, bg1_ref, wg2_ref,
               bg2_ref, convr_ref, wxt_ref, bxt_ref, embs_ref, wp_ref,
               ap_ref, wfg_ref, bfg_ref, wf1_ref, bf1_ref, wf2_ref,
               bf2_ref, wo_ref, bo_ref, o_ref, xg_scr):
    act3 = jnp.maximum(agg3_ref[:, :] + b3_ref[:, :], 0.0)
    bcol = batch_ref[:, :]

    def body(g, _):
        mask = bcol == g
        masked = jnp.where(mask, act3, -jnp.inf)
        xg_scr[pl.ds(g, 1), :] = jnp.max(masked, axis=0, keepdims=True)
        return 0

    jax.lax.fori_loop(0, B, body, 0)
    xg = xg_scr[:, :]
    xg = jnp.maximum(jnp.dot(xg, wg1_ref[:, :],
                             preferred_element_type=jnp.float32)
                     + bg1_ref[:, :], 0.0)
    xg = jnp.dot(xg, wg2_ref[:, :],
                 preferred_element_type=jnp.float32) + bg2_ref[:, :]

    xt = jnp.dot(convr_ref[:, :], wxt_ref[:, :],
                 preferred_element_type=jnp.float32) + bxt_ref[:, :]

    hp = jnp.maximum(jnp.dot(embs_ref[:, :], wp_ref[:, :],
                             preferred_element_type=jnp.float32), 0.0)
    logits = jnp.dot(hp, ap_ref[:, :],
                     preferred_element_type=jnp.float32)
    e = jnp.exp(logits)
    rwho = jax.lax.broadcasted_iota(jnp.int32, (B, 13 * B), 1) // 13
    gsel = (rwho == jax.lax.broadcasted_iota(jnp.int32, (B, 13 * B), 0)
            ).astype(jnp.float32)
    denom = jnp.dot(gsel, e, preferred_element_type=jnp.float32)
    nwho = jax.lax.broadcasted_iota(jnp.int32, (13 * B, B), 0) // 13
    gselT = (nwho == jax.lax.broadcasted_iota(jnp.int32, (13 * B, B), 1)
             ).astype(jnp.float32)
    denom_n = jnp.dot(gselT, denom, preferred_element_type=jnp.float32)
    attf = e / denom_n
    fg = jnp.dot(gsel, hp * attf,
                 preferred_element_type=jnp.float32)
    ofg = jnp.dot(fg, wfg_ref[:, :],
                  preferred_element_type=jnp.float32) + bfg_ref[:, :]

    xc = jnp.concatenate([xg, xt, ofg], axis=1)
    xc = jnp.maximum(jnp.dot(xc, wf1_ref[:, :],
                             preferred_element_type=jnp.float32)
                     + bf1_ref[:, :], 0.0)
    xc = jnp.maximum(jnp.dot(xc, wf2_ref[:, :],
                             preferred_element_type=jnp.float32)
                     + bf2_ref[:, :], 0.0)
    o_ref[:, :] = jnp.dot(xc, wo_ref[:, :],
                          preferred_element_type=jnp.float32) + bo_ref[:, :]


def _head(agg3, batch_col, b3r, Wg1, bg1r, Wg2, bg2r, convr, W_xt, bxtr,
          embs, Wp, apc, Wfg, bfgr, Wf1, bf1r, Wf2, bf2r, Wo, bor):
    return pl.pallas_call(
        _head_kern,
        out_shape=jax.ShapeDtypeStruct((B, 1), jnp.float32),
        scratch_shapes=[pltpu.VMEM((B, 312), jnp.float32)],
    )(agg3, batch_col, b3r, Wg1, bg1r, Wg2, bg2r, convr, W_xt, bxtr,
      embs, Wp, apc, Wfg, bfgr, Wf1, bf1r, Wf2, bf2r, Wo, bor)


@jax.jit
def _run(x, edge_index, batch, target, embs, W1, b1, W2, b2, W3, b3,
         Wg1, bg1, Wg2, bg2, emb_xt, k_xt, bc_xt, W_xt, b_xt, Wp, a_p,
         Wfg, bfg, Wf1, bf1, Wf2, bf2, Wo, bo):
    src, dst = edge_index[0], edge_index[1]
    loop = jnp.arange(N, dtype=src.dtype)
    s2 = jnp.concatenate([src, loop])
    d2 = jnp.concatenate([dst, loop])
    perm = jnp.argsort(d2)
    ds = d2[perm]
    ss = s2[perm]
    pad = E2P - E2
    ds_p = jnp.concatenate([ds, jnp.full((pad,), N - 1, jnp.int32)])
    ss_p = jnp.concatenate([ss, jnp.zeros((pad,), jnp.int32)])
    valid = (jnp.arange(E2P) < E2).astype(jnp.float32)[:, None]
    ds3 = ds_p.reshape(NBLK, 1, BLK)
    dmins = (ds_p[::BLK] // 8) * 8

    deg = _scatter(dmins, valid, valid, ds3, 1)[:N, :]
    dinv = jax.lax.rsqrt(deg[:, 0])
    coeff = (dinv[ss_p] * dinv[ds_p])[:, None] * valid

    h = _mm(x, W1)
    agg1 = _scatter(dmins, h[ss_p], coeff, ds3, 78)[:N, :]
    h = _mm_bias_relu(agg1, b1[None, :], W2)
    agg2 = _scatter(dmins, h[ss_p], coeff, ds3, 156)[:N, :]
    h = _mm_bias_relu(agg2, b2[None, :], W3)
    agg3 = _scatter(dmins, h[ss_p], coeff, ds3, 312)[:N, :]

    et = jnp.take(emb_xt, target, axis=0)
    convr = _conv(et, k_xt, bc_xt[:, None]).reshape(B, 32 * 121)

    out = _head(agg3, batch[:, None].astype(jnp.int32), b3[None, :],
                Wg1, bg1[None, :], Wg2, bg2[None, :], convr, W_xt,
                b_xt[None, :], embs, Wp, a_p[:, None], Wfg, bfg[None, :],
                Wf1, bf1[None, :], Wf2, bf2[None, :], Wo, bo[None, :])
    return out


def kernel(x, edge_index, batch, target, embs, W1, b1, W2, b2, W3, b3,
           Wg1, bg1, Wg2, bg2, emb_xt, k_xt, bc_xt, W_xt, b_xt, Wp, a_p,
           Wfg, bfg, Wf1, bf1, Wf2, bf2, Wo, bo):
    return _run(x, edge_index, batch, target, embs, W1, b1, W2, b2, W3,
                b3, Wg1, bg1, Wg2, bg2, emb_xt, k_xt, bc_xt, W_xt, b_xt,
                Wp, a_p, Wfg, bfg, Wf1, bf1, Wf2, bf2, Wo, bo)
